# baseline (speedup 1.0000x reference)
import functools

import jax
import jax.numpy as jnp
from jax import lax
from jax.experimental import pallas as pl
from jax.experimental.pallas import tpu as pltpu
from jax.experimental.pallas import tpu_sc as plsc

_B, _L, _C, _H, _W = 2, 24, 8, 64, 64
_HW = _H * _W
_CHW = _C * _HW
_FHW = 2 * _HW
_NSLOT = _B * _L
_NC, _NS = 2, 16
_NW = _NC * _NS
_NGRP = _HW // 16


def _schedule():
    import bisect

    _OH = 0.4
    jobs = [[] for _ in range(_NW)]
    merge = [(-1, -1, -1)] * _NW
    npart = 0
    for b in range(_B):
        stream = [(t, k) for t in range(_L) for k in range(t + 1)]
        cum = []
        c = 0.0
        for (t, k) in stream:
            c += 1.0 + (_OH if k == 0 else 0.0)
            cum.append(c)
        cuts = [0]
        for i in range(1, _NS):
            cuts.append(bisect.bisect_left(cum, i * c / _NS))
        cuts.append(len(stream))
        partials = {}
        for ci in range(_NS):
            part = stream[cuts[ci]:cuts[ci + 1]]
            wid = ci * _NC + b
            by_t = {}
            for (t, k) in part:
                by_t.setdefault(t, []).append(k)
            for t in sorted(by_t):
                ks = by_t[t]
                k0, k1 = ks[0], ks[-1] + 1
                assert ks == list(range(k0, k1))
                slot = b * _L + t
                if k0 == 0:
                    jobs[wid].append((slot, k0, k1, slot))
                else:
                    pslot = _NSLOT + 1 + npart
                    npart += 1
                    jobs[wid].append((slot, k0, k1, pslot))
                    partials.setdefault(slot, []).append(pslot)
        mcount = 0
        for slot in sorted(partials):
            ps = partials[slot]
            assert len(ps) <= 2, "slot split into >3 segments"
            mwid = (mcount % _NS) * _NC + b
            assert merge[mwid] == (-1, -1, -1)
            merge[mwid] = (slot, ps[0], ps[1] if len(ps) > 1 else -1)
            mcount += 1
        assert mcount <= _NS
    maxj = max(len(j) for j in jobs)
    tbl = []
    for i in range(_NW):
        row = list(jobs[i]) + [(-1, 0, 0, 0)] * (maxj - len(jobs[i]))
        tbl.append(row)
    return tbl, maxj, merge, npart


_TBL, _MAXJ, _MERGE, _NPART = _schedule()
_NOUT = _NSLOT + 1 + _NPART


def _sel(wid, vals):
    r = jnp.int32(vals[0])
    for i in range(1, _NW):
        r = jnp.where(wid == i, jnp.int32(vals[i]), r)
    return r


_GCH = 32
_GV = 16 * _GCH


def _sc_body(images_hbm, cumt_hbm, cumk_hbm, out_hbm,
             img_v, acc_v, ft_v, fk_v, ai_v, wf_v, sem_i, sem_f):
    cid = lax.axis_index("c")
    sid = lax.axis_index("s")
    wid = sid * _NC + cid

    def make_phase_a(fbase, g0):
      def phase_a(i):
        gl = g0 * 16 + i * 16
        gx = ft_v[pl.ds(gl, 16)] - fk_v[pl.ds(fbase + gl, 16)]
        gy = ft_v[pl.ds(_HW + gl, 16)] - fk_v[pl.ds(fbase + _HW + gl, 16)]
        r = lax.rem(gx + 1.0, 2.0)
        r = jnp.where(r < 0.0, r + 2.0, r)
        xs = r * jnp.float32(_W / 2) + 0.5
        xi = xs.astype(jnp.int32)
        wx1 = xs - xi.astype(jnp.float32)
        wx0 = 1.0 - wx1
        y = gy * jnp.float32(_H / 2) + jnp.float32(_H / 2 - 0.5)
        yi = y.astype(jnp.int32)
        y0 = yi - jnp.where(y < yi.astype(jnp.float32), 1, 0)
        wy1 = y - y0.astype(jnp.float32)
        wy0 = 1.0 - wy1
        ax0 = jnp.where(xi >= 1, wx0, 0.0)
        ax1 = jnp.where(xi <= _W - 1, wx1, 0.0)
        ay0 = jnp.where((y0 >= 0) & (y0 <= _H - 1), wy0, 0.0)
        ay1 = jnp.where((y0 >= -1) & (y0 <= _H - 2), wy1, 0.0)
        x0c = jnp.maximum(xi - 1, 0)
        x1c = jnp.minimum(xi, _W - 1)
        y0c = jnp.minimum(jnp.maximum(y0, 0), _H - 1)
        y1c = jnp.minimum(jnp.maximum(y0 + 1, 0), _H - 1)
        row0 = y0c * _W
        row1 = y1c * _W
        sl = pl.ds(i * 16, 16)
        ai_v[sl] = row0 + x0c
        ai_v[pl.ds(_GV + i * 16, 16)] = row0 + x1c
        ai_v[pl.ds(2 * _GV + i * 16, 16)] = row1 + x0c
        ai_v[pl.ds(3 * _GV + i * 16, 16)] = row1 + x1c
        wf_v[sl] = ax0 * ay0
        wf_v[pl.ds(_GV + i * 16, 16)] = ax1 * ay0
        wf_v[pl.ds(2 * _GV + i * 16, 16)] = ax0 * ay1
        wf_v[pl.ds(3 * _GV + i * 16, 16)] = ax1 * ay1
      return phase_a

    def make_phase_b(ibase, g0, store):
      def phase_b(i):
        gl = g0 * 16 + i * 16
        il = i * 16
        a00 = ai_v[pl.ds(il, 16)]
        a10 = ai_v[pl.ds(_GV + il, 16)]
        a01 = ai_v[pl.ds(2 * _GV + il, 16)]
        a11 = ai_v[pl.ds(3 * _GV + il, 16)]
        w00 = wf_v[pl.ds(il, 16)]
        w10 = wf_v[pl.ds(_GV + il, 16)]
        w01 = wf_v[pl.ds(2 * _GV + il, 16)]
        w11 = wf_v[pl.ds(3 * _GV + il, 16)]
        for c in range(_C):
            ref_c = img_v.at[pl.ds(ibase + c * _HW, _HW)]
            v0 = (plsc.load_gather(ref_c, [a00]) * w00
                  + plsc.load_gather(ref_c, [a10]) * w10)
            v1 = (plsc.load_gather(ref_c, [a01]) * w01
                  + plsc.load_gather(ref_c, [a11]) * w11)
            v = v0 + v1
            sl = pl.ds(c * _HW + gl, 16)
            if store:
                acc_v[sl] = v
            else:
                acc_v[sl] = acc_v[sl] + v
      return phase_b

    for j in range(_MAXJ):
        slot = _sel(wid, [_TBL[i][j][0] for i in range(_NW)])
        k0 = _sel(wid, [_TBL[i][j][1] for i in range(_NW)])
        nk = _sel(wid, [_TBL[i][j][2] - _TBL[i][j][1] for i in range(_NW)])
        dest = _sel(wid, [_TBL[i][j][3] for i in range(_NW)])

        @pl.when(slot >= 0)
        def _job():
            b = slot // _L

            pltpu.sync_copy(cumt_hbm.at[pl.ds(slot * _FHW, _FHW)], ft_v)

            src0 = b * _L + k0
            pltpu.async_copy(images_hbm.at[pl.ds(src0 * _CHW, _CHW)],
                             img_v.at[pl.ds(0, _CHW)], sem_i.at[0])
            pltpu.async_copy(cumk_hbm.at[pl.ds(src0 * _FHW, _FHW)],
                             fk_v.at[pl.ds(0, _FHW)], sem_f.at[0])

            def k_body(i, _):
                par = jnp.bitwise_and(i, 1)
                ibase = par * _CHW
                fbase = par * _FHW
                src = src0 + i
                pltpu.make_async_copy(
                    images_hbm.at[pl.ds(src * _CHW, _CHW)],
                    img_v.at[pl.ds(ibase, _CHW)], sem_i.at[par]).wait()
                pltpu.make_async_copy(
                    cumk_hbm.at[pl.ds(src * _FHW, _FHW)],
                    fk_v.at[pl.ds(fbase, _FHW)], sem_f.at[par]).wait()

                @pl.when(i + 1 < nk)
                def _issue_next():
                    npar = 1 - par
                    pltpu.async_copy(
                        images_hbm.at[pl.ds((src + 1) * _CHW, _CHW)],
                        img_v.at[pl.ds(npar * _CHW, _CHW)], sem_i.at[npar])
                    pltpu.async_copy(
                        cumk_hbm.at[pl.ds((src + 1) * _FHW, _FHW)],
                        fk_v.at[pl.ds(npar * _FHW, _FHW)], sem_f.at[npar])

                @pl.when(i == 0)
                def _first():
                    def chunk0(cb, _2):
                        g0 = cb * _GCH
                        plsc.parallel_loop(0, _GCH, 1, unroll=2)(
                            make_phase_a(0, g0))
                        plsc.parallel_loop(0, _GCH, 1, unroll=1)(
                            make_phase_b(0, g0, True))
                        return _2
                    lax.fori_loop(0, _NGRP // _GCH, chunk0, 0)

                @pl.when(i > 0)
                def _rest():
                    def chunkn(cb, _2):
                        g0 = cb * _GCH
                        plsc.parallel_loop(0, _GCH, 1, unroll=2)(
                            make_phase_a(fbase, g0))
                        plsc.parallel_loop(0, _GCH, 1, unroll=1)(
                            make_phase_b(ibase, g0, False))
                        return _2
                    lax.fori_loop(0, _NGRP // _GCH, chunkn, 0)

                return _

            lax.fori_loop(0, nk, k_body, 0)

            pltpu.sync_copy(acc_v, out_hbm.at[pl.ds(dest * _CHW, _CHW)])

    plsc.subcore_barrier()
    mslot = _sel(wid, [_MERGE[i][0] for i in range(_NW)])
    pslot = _sel(wid, [_MERGE[i][1] for i in range(_NW)])
    pslot2 = _sel(wid, [_MERGE[i][2] for i in range(_NW)])

    @pl.when(mslot >= 0)
    def _merge_job():
        pltpu.async_copy(out_hbm.at[pl.ds(mslot * _CHW, _CHW)],
                         img_v.at[pl.ds(0, _CHW)], sem_i.at[0])
        pltpu.async_copy(out_hbm.at[pl.ds(pslot * _CHW, _CHW)],
                         img_v.at[pl.ds(_CHW, _CHW)], sem_i.at[1])
        pltpu.make_async_copy(out_hbm.at[pl.ds(mslot * _CHW, _CHW)],
                              img_v.at[pl.ds(0, _CHW)], sem_i.at[0]).wait()
        pltpu.make_async_copy(out_hbm.at[pl.ds(pslot * _CHW, _CHW)],
                              img_v.at[pl.ds(_CHW, _CHW)], sem_i.at[1]).wait()

        @plsc.parallel_loop(0, _CHW // 16, 1, unroll=4)
        def _madd(i):
            gl = i * 16
            img_v[pl.ds(gl, 16)] = (img_v[pl.ds(gl, 16)]
                                    + img_v[pl.ds(_CHW + gl, 16)])

        @pl.when(pslot2 >= 0)
        def _second_partial():
            pltpu.sync_copy(out_hbm.at[pl.ds(pslot2 * _CHW, _CHW)],
                            img_v.at[pl.ds(_CHW, _CHW)])

            @plsc.parallel_loop(0, _CHW // 16, 1, unroll=4)
            def _madd2(i):
                gl = i * 16
                img_v[pl.ds(gl, 16)] = (img_v[pl.ds(gl, 16)]
                                        + img_v[pl.ds(_CHW + gl, 16)])

        pltpu.sync_copy(img_v.at[pl.ds(0, _CHW)],
                        out_hbm.at[pl.ds(mslot * _CHW, _CHW)])


_mesh = plsc.VectorSubcoreMesh(core_axis_name="c", subcore_axis_name="s",
                               num_cores=_NC, num_subcores=_NS)

_sc_call = functools.partial(
    pl.kernel,
    mesh=_mesh,
    compiler_params=pltpu.CompilerParams(use_tc_tiling_on_sc=False,
                                         needs_layout_passes=False),
    out_type=jax.ShapeDtypeStruct((_NOUT * _CHW,), jnp.float32),
    scratch_types=[
        pltpu.VMEM((2 * _CHW,), jnp.float32),
        pltpu.VMEM((_CHW,), jnp.float32),
        pltpu.VMEM((_FHW,), jnp.float32),
        pltpu.VMEM((2 * _FHW,), jnp.float32),
        pltpu.VMEM((4 * _GV,), jnp.int32),
        pltpu.VMEM((4 * _GV,), jnp.float32),
        pltpu.SemaphoreType.DMA((2,)),
        pltpu.SemaphoreType.DMA((2,)),
    ],
)(_sc_body)


def kernel(flows, images):
    dtype = flows.dtype
    cum = jnp.cumsum(flows.astype(jnp.float32), axis=1).astype(dtype)
    sy = 2.0 / _H
    sx = 2.0 / _W
    gyc = jnp.linspace(-1.0 + sy * 0.5, 1.0 - sy * 0.5, _H, dtype=dtype)
    gxc = jnp.linspace(-1.0 + sx * 0.5, 1.0 - sx * 0.5, _W, dtype=dtype)
    bx = jnp.tile(gxc, _H)
    by = jnp.repeat(gyc, _W)
    base = jnp.stack([bx, by])
    cumf = cum.reshape(_B, _L, 2, _HW)
    cumt = (cumf + base[None, None]).reshape(-1)
    out_flat = _sc_call(images.reshape(-1), cumt, cumf.reshape(-1))
    return out_flat[: _NSLOT * _CHW].reshape(_B, _L, _C, _H, _W)


# speedup vs baseline: 1.3384x; 1.3384x over previous
import functools

import jax
import jax.numpy as jnp
from jax import lax
from jax.experimental import pallas as pl
from jax.experimental.pallas import tpu as pltpu
from jax.experimental.pallas import tpu_sc as plsc

_B, _L, _C, _H, _W = 2, 24, 8, 64, 64
_HW = _H * _W
_CHW = _C * _HW
_FHW = 2 * _HW
_NSLOT = _B * _L
_NC, _NS = 2, 16
_NW = _NC * _NS
_NGRP = _HW // 16


def _schedule():
    import bisect

    _OH = 0.25
    jobs = [[] for _ in range(_NW)]
    merge = [(-1, -1, -1)] * _NW
    npart = 0
    for b in range(_B):
        stream = [(t, k) for t in range(_L) for k in range(t + 1)]
        cum = []
        c = 0.0
        for (t, k) in stream:
            c += 1.0 + (_OH if k == 0 else 0.0)
            cum.append(c)
        cuts = [0]
        for i in range(1, _NS):
            cuts.append(bisect.bisect_left(cum, i * c / _NS))
        cuts.append(len(stream))
        partials = {}
        for ci in range(_NS):
            part = stream[cuts[ci]:cuts[ci + 1]]
            wid = ci * _NC + b
            by_t = {}
            for (t, k) in part:
                by_t.setdefault(t, []).append(k)
            for t in sorted(by_t):
                ks = by_t[t]
                k0, k1 = ks[0], ks[-1] + 1
                assert ks == list(range(k0, k1))
                slot = b * _L + t
                if k0 == 0:
                    jobs[wid].append((slot, k0, k1, slot))
                else:
                    pslot = _NSLOT + 1 + npart
                    npart += 1
                    jobs[wid].append((slot, k0, k1, pslot))
                    partials.setdefault(slot, []).append(pslot)
        mcount = 0
        for slot in sorted(partials):
            ps = partials[slot]
            assert len(ps) <= 2, "slot split into >3 segments"
            mwid = (mcount % _NS) * _NC + b
            assert merge[mwid] == (-1, -1, -1)
            merge[mwid] = (slot, ps[0], ps[1] if len(ps) > 1 else -1)
            mcount += 1
        assert mcount <= _NS
    maxj = max(len(j) for j in jobs)
    tbl = []
    for i in range(_NW):
        row = list(jobs[i]) + [(-1, 0, 0, 0)] * (maxj - len(jobs[i]))
        tbl.append(row)
    return tbl, maxj, merge, npart


_TBL, _MAXJ, _MERGE, _NPART = _schedule()
_NOUT = _NSLOT + 1 + _NPART


def _sel(wid, vals):
    r = jnp.int32(vals[0])
    for i in range(1, _NW):
        r = jnp.where(wid == i, jnp.int32(vals[i]), r)
    return r


def _sc_body(images_hbm, cumt_hbm, cumk_hbm, out_hbm,
             img_v, acc_v, ft_v, fk_v, sem_i, sem_f):
    cid = lax.axis_index("c")
    sid = lax.axis_index("s")
    wid = sid * _NC + cid

    def make_group_body(ibase, fbase, store):
      def group_body(g):
        gl = g * 16
        gx = ft_v[pl.ds(gl, 16)] - fk_v[pl.ds(fbase + gl, 16)]
        gy = ft_v[pl.ds(_HW + gl, 16)] - fk_v[pl.ds(fbase + _HW + gl, 16)]
        r = lax.rem(gx + 1.0, 2.0)
        r = jnp.where(r < 0.0, r + 2.0, r)
        xs = r * jnp.float32(_W / 2) + 0.5
        xi = xs.astype(jnp.int32)
        wx1 = xs - xi.astype(jnp.float32)
        wx0 = 1.0 - wx1
        y = gy * jnp.float32(_H / 2) + jnp.float32(_H / 2 - 0.5)
        yi = y.astype(jnp.int32)
        y0 = yi - jnp.where(y < yi.astype(jnp.float32), 1, 0)
        wy1 = y - y0.astype(jnp.float32)
        wy0 = 1.0 - wy1
        ax0 = jnp.where(xi >= 1, wx0, 0.0)
        ax1 = jnp.where(xi <= _W - 1, wx1, 0.0)
        ay0 = jnp.where((y0 >= 0) & (y0 <= _H - 1), wy0, 0.0)
        ay1 = jnp.where((y0 >= -1) & (y0 <= _H - 2), wy1, 0.0)
        x0c = jnp.maximum(xi - 1, 0)
        x1c = jnp.minimum(xi, _W - 1)
        y0c = jnp.minimum(jnp.maximum(y0, 0), _H - 1)
        y1c = jnp.minimum(jnp.maximum(y0 + 1, 0), _H - 1)
        row0 = y0c * _W
        row1 = y1c * _W
        a00 = row0 + x0c
        a10 = row0 + x1c
        a01 = row1 + x0c
        a11 = row1 + x1c
        w00 = ax0 * ay0
        w10 = ax1 * ay0
        w01 = ax0 * ay1
        w11 = ax1 * ay1
        for c in range(_C):
            ref_c = img_v.at[pl.ds(ibase + c * _HW, _HW)]
            v0 = (plsc.load_gather(ref_c, [a00]) * w00
                  + plsc.load_gather(ref_c, [a10]) * w10)
            v1 = (plsc.load_gather(ref_c, [a01]) * w01
                  + plsc.load_gather(ref_c, [a11]) * w11)
            v = v0 + v1
            sl = pl.ds(c * _HW + gl, 16)
            if store:
                acc_v[sl] = v
            else:
                acc_v[sl] = acc_v[sl] + v
      return group_body

    for j in range(_MAXJ):
        slot = _sel(wid, [_TBL[i][j][0] for i in range(_NW)])
        k0 = _sel(wid, [_TBL[i][j][1] for i in range(_NW)])
        nk = _sel(wid, [_TBL[i][j][2] - _TBL[i][j][1] for i in range(_NW)])
        dest = _sel(wid, [_TBL[i][j][3] for i in range(_NW)])

        @pl.when(slot >= 0)
        def _job():
            b = slot // _L

            pltpu.sync_copy(cumt_hbm.at[pl.ds(slot * _FHW, _FHW)], ft_v)

            src0 = b * _L + k0
            pltpu.async_copy(images_hbm.at[pl.ds(src0 * _CHW, _CHW)],
                             img_v.at[pl.ds(0, _CHW)], sem_i.at[0])
            pltpu.async_copy(cumk_hbm.at[pl.ds(src0 * _FHW, _FHW)],
                             fk_v.at[pl.ds(0, _FHW)], sem_f.at[0])

            def k_body(i, _):
                par = jnp.bitwise_and(i, 1)
                ibase = par * _CHW
                fbase = par * _FHW
                src = src0 + i
                pltpu.make_async_copy(
                    images_hbm.at[pl.ds(src * _CHW, _CHW)],
                    img_v.at[pl.ds(ibase, _CHW)], sem_i.at[par]).wait()
                pltpu.make_async_copy(
                    cumk_hbm.at[pl.ds(src * _FHW, _FHW)],
                    fk_v.at[pl.ds(fbase, _FHW)], sem_f.at[par]).wait()

                @pl.when(i + 1 < nk)
                def _issue_next():
                    npar = 1 - par
                    pltpu.async_copy(
                        images_hbm.at[pl.ds((src + 1) * _CHW, _CHW)],
                        img_v.at[pl.ds(npar * _CHW, _CHW)], sem_i.at[npar])
                    pltpu.async_copy(
                        cumk_hbm.at[pl.ds((src + 1) * _FHW, _FHW)],
                        fk_v.at[pl.ds(npar * _FHW, _FHW)], sem_f.at[npar])

                @pl.when(i == 0)
                def _first():
                    plsc.parallel_loop(0, _NGRP, 1, unroll=1)(
                        make_group_body(0, 0, True))

                @pl.when(i > 0)
                def _rest():
                    plsc.parallel_loop(0, _NGRP, 1, unroll=1)(
                        make_group_body(ibase, fbase, False))

                return _

            lax.fori_loop(0, nk, k_body, 0)

            pltpu.sync_copy(acc_v, out_hbm.at[pl.ds(dest * _CHW, _CHW)])

    plsc.subcore_barrier()
    mslot = _sel(wid, [_MERGE[i][0] for i in range(_NW)])
    pslot = _sel(wid, [_MERGE[i][1] for i in range(_NW)])
    pslot2 = _sel(wid, [_MERGE[i][2] for i in range(_NW)])

    @pl.when(mslot >= 0)
    def _merge_job():
        pltpu.async_copy(out_hbm.at[pl.ds(mslot * _CHW, _CHW)],
                         img_v.at[pl.ds(0, _CHW)], sem_i.at[0])
        pltpu.async_copy(out_hbm.at[pl.ds(pslot * _CHW, _CHW)],
                         img_v.at[pl.ds(_CHW, _CHW)], sem_i.at[1])
        pltpu.make_async_copy(out_hbm.at[pl.ds(mslot * _CHW, _CHW)],
                              img_v.at[pl.ds(0, _CHW)], sem_i.at[0]).wait()
        pltpu.make_async_copy(out_hbm.at[pl.ds(pslot * _CHW, _CHW)],
                              img_v.at[pl.ds(_CHW, _CHW)], sem_i.at[1]).wait()

        @plsc.parallel_loop(0, _CHW // 16, 1, unroll=4)
        def _madd(i):
            gl = i * 16
            img_v[pl.ds(gl, 16)] = (img_v[pl.ds(gl, 16)]
                                    + img_v[pl.ds(_CHW + gl, 16)])

        @pl.when(pslot2 >= 0)
        def _second_partial():
            pltpu.sync_copy(out_hbm.at[pl.ds(pslot2 * _CHW, _CHW)],
                            img_v.at[pl.ds(_CHW, _CHW)])

            @plsc.parallel_loop(0, _CHW // 16, 1, unroll=4)
            def _madd2(i):
                gl = i * 16
                img_v[pl.ds(gl, 16)] = (img_v[pl.ds(gl, 16)]
                                        + img_v[pl.ds(_CHW + gl, 16)])

        pltpu.sync_copy(img_v.at[pl.ds(0, _CHW)],
                        out_hbm.at[pl.ds(mslot * _CHW, _CHW)])


_mesh = plsc.VectorSubcoreMesh(core_axis_name="c", subcore_axis_name="s",
                               num_cores=_NC, num_subcores=_NS)

_sc_call = functools.partial(
    pl.kernel,
    mesh=_mesh,
    compiler_params=pltpu.CompilerParams(use_tc_tiling_on_sc=False,
                                         needs_layout_passes=False),
    out_type=jax.ShapeDtypeStruct((_NOUT * _CHW,), jnp.float32),
    scratch_types=[
        pltpu.VMEM((2 * _CHW,), jnp.float32),
        pltpu.VMEM((_CHW,), jnp.float32),
        pltpu.VMEM((_FHW,), jnp.float32),
        pltpu.VMEM((2 * _FHW,), jnp.float32),
        pltpu.SemaphoreType.DMA((2,)),
        pltpu.SemaphoreType.DMA((2,)),
    ],
)(_sc_body)


def kernel(flows, images):
    dtype = flows.dtype
    cum = jnp.cumsum(flows.astype(jnp.float32), axis=1).astype(dtype)
    sy = 2.0 / _H
    sx = 2.0 / _W
    gyc = jnp.linspace(-1.0 + sy * 0.5, 1.0 - sy * 0.5, _H, dtype=dtype)
    gxc = jnp.linspace(-1.0 + sx * 0.5, 1.0 - sx * 0.5, _W, dtype=dtype)
    bx = jnp.tile(gxc, _H)
    by = jnp.repeat(gyc, _W)
    base = jnp.stack([bx, by])
    cumf = cum.reshape(_B, _L, 2, _HW)
    cumt = (cumf + base[None, None]).reshape(-1)
    out_flat = _sc_call(images.reshape(-1), cumt, cumf.reshape(-1))
    return out_flat[: _NSLOT * _CHW].reshape(_B, _L, _C, _H, _W)


# speedup vs baseline: 1.3420x; 1.0027x over previous
import functools

import jax
import jax.numpy as jnp
from jax import lax
from jax.experimental import pallas as pl
from jax.experimental.pallas import tpu as pltpu
from jax.experimental.pallas import tpu_sc as plsc

_B, _L, _C, _H, _W = 2, 24, 8, 64, 64
_HW = _H * _W
_CHW = _C * _HW
_FHW = 2 * _HW
_NSLOT = _B * _L
_NC, _NS = 2, 16
_NW = _NC * _NS
_NGRP = _HW // 16


def _schedule():
    import bisect

    _OH = 0.4
    jobs = [[] for _ in range(_NW)]
    merge = [(-1, -1, -1)] * _NW
    npart = 0
    for b in range(_B):
        stream = [(t, k) for t in range(_L) for k in range(t + 1)]
        cum = []
        c = 0.0
        for (t, k) in stream:
            c += 1.0 + (_OH if k == 0 else 0.0)
            cum.append(c)
        cuts = [0]
        for i in range(1, _NS):
            cuts.append(bisect.bisect_left(cum, i * c / _NS))
        cuts.append(len(stream))
        partials = {}
        for ci in range(_NS):
            part = stream[cuts[ci]:cuts[ci + 1]]
            wid = ci * _NC + b
            by_t = {}
            for (t, k) in part:
                by_t.setdefault(t, []).append(k)
            for t in sorted(by_t):
                ks = by_t[t]
                k0, k1 = ks[0], ks[-1] + 1
                assert ks == list(range(k0, k1))
                slot = b * _L + t
                if k0 == 0:
                    jobs[wid].append((slot, k0, k1, slot))
                else:
                    pslot = _NSLOT + 1 + npart
                    npart += 1
                    jobs[wid].append((slot, k0, k1, pslot))
                    partials.setdefault(slot, []).append(pslot)
        mcount = 0
        for slot in sorted(partials):
            ps = partials[slot]
            assert len(ps) <= 2, "slot split into >3 segments"
            mwid = (mcount % _NS) * _NC + b
            assert merge[mwid] == (-1, -1, -1)
            merge[mwid] = (slot, ps[0], ps[1] if len(ps) > 1 else -1)
            mcount += 1
        assert mcount <= _NS
    maxj = max(len(j) for j in jobs)
    tbl = []
    for i in range(_NW):
        row = list(jobs[i]) + [(-1, 0, 0, 0)] * (maxj - len(jobs[i]))
        tbl.append(row)
    return tbl, maxj, merge, npart


_TBL, _MAXJ, _MERGE, _NPART = _schedule()
_NOUT = _NSLOT + 1 + _NPART


def _sel(wid, vals):
    r = jnp.int32(vals[0])
    for i in range(1, _NW):
        r = jnp.where(wid == i, jnp.int32(vals[i]), r)
    return r


def _sc_body(images_hbm, cumt_hbm, cumk_hbm, out_hbm,
             img_v, acc_v, ft_v, fk_v, sem_i, sem_f):
    cid = lax.axis_index("c")
    sid = lax.axis_index("s")
    wid = sid * _NC + cid

    def make_group_body(ibase, fbase, store):
      def group_body(g):
        gl = g * 16
        gx = ft_v[pl.ds(gl, 16)] - fk_v[pl.ds(fbase + gl, 16)]
        gy = ft_v[pl.ds(_HW + gl, 16)] - fk_v[pl.ds(fbase + _HW + gl, 16)]
        r = lax.rem(gx + 1.0, 2.0)
        r = jnp.where(r < 0.0, r + 2.0, r)
        xs = r * jnp.float32(_W / 2) + 0.5
        xi = xs.astype(jnp.int32)
        wx1 = xs - xi.astype(jnp.float32)
        wx0 = 1.0 - wx1
        y = gy * jnp.float32(_H / 2) + jnp.float32(_H / 2 - 0.5)
        yi = y.astype(jnp.int32)
        y0 = yi - jnp.where(y < yi.astype(jnp.float32), 1, 0)
        wy1 = y - y0.astype(jnp.float32)
        wy0 = 1.0 - wy1
        ax0 = jnp.where(xi >= 1, wx0, 0.0)
        ax1 = jnp.where(xi <= _W - 1, wx1, 0.0)
        ay0 = jnp.where((y0 >= 0) & (y0 <= _H - 1), wy0, 0.0)
        ay1 = jnp.where((y0 >= -1) & (y0 <= _H - 2), wy1, 0.0)
        x0c = jnp.maximum(xi - 1, 0)
        x1c = jnp.minimum(xi, _W - 1)
        y0c = jnp.minimum(jnp.maximum(y0, 0), _H - 1)
        y1c = jnp.minimum(jnp.maximum(y0 + 1, 0), _H - 1)
        row0 = y0c * _W
        row1 = y1c * _W
        a00 = row0 + x0c
        a10 = row0 + x1c
        a01 = row1 + x0c
        a11 = row1 + x1c
        w00 = ax0 * ay0
        w10 = ax1 * ay0
        w01 = ax0 * ay1
        w11 = ax1 * ay1
        for c in range(_C):
            ref_c = img_v.at[pl.ds(ibase + c * _HW, _HW)]
            v0 = (plsc.load_gather(ref_c, [a00]) * w00
                  + plsc.load_gather(ref_c, [a10]) * w10)
            v1 = (plsc.load_gather(ref_c, [a01]) * w01
                  + plsc.load_gather(ref_c, [a11]) * w11)
            v = v0 + v1
            sl = pl.ds(c * _HW + gl, 16)
            if store:
                acc_v[sl] = v
            else:
                acc_v[sl] = acc_v[sl] + v
      return group_body

    for j in range(_MAXJ):
        slot = _sel(wid, [_TBL[i][j][0] for i in range(_NW)])
        k0 = _sel(wid, [_TBL[i][j][1] for i in range(_NW)])
        nk = _sel(wid, [_TBL[i][j][2] - _TBL[i][j][1] for i in range(_NW)])
        dest = _sel(wid, [_TBL[i][j][3] for i in range(_NW)])

        @pl.when(slot >= 0)
        def _job():
            b = slot // _L

            pltpu.sync_copy(cumt_hbm.at[pl.ds(slot * _FHW, _FHW)], ft_v)

            src0 = b * _L + k0
            pltpu.async_copy(images_hbm.at[pl.ds(src0 * _CHW, _CHW)],
                             img_v.at[pl.ds(0, _CHW)], sem_i.at[0])
            pltpu.async_copy(cumk_hbm.at[pl.ds(src0 * _FHW, _FHW)],
                             fk_v.at[pl.ds(0, _FHW)], sem_f.at[0])

            def k_body(i, _):
                par = jnp.bitwise_and(i, 1)
                ibase = par * _CHW
                fbase = par * _FHW
                src = src0 + i
                pltpu.make_async_copy(
                    images_hbm.at[pl.ds(src * _CHW, _CHW)],
                    img_v.at[pl.ds(ibase, _CHW)], sem_i.at[par]).wait()
                pltpu.make_async_copy(
                    cumk_hbm.at[pl.ds(src * _FHW, _FHW)],
                    fk_v.at[pl.ds(fbase, _FHW)], sem_f.at[par]).wait()

                @pl.when(i + 1 < nk)
                def _issue_next():
                    npar = 1 - par
                    pltpu.async_copy(
                        images_hbm.at[pl.ds((src + 1) * _CHW, _CHW)],
                        img_v.at[pl.ds(npar * _CHW, _CHW)], sem_i.at[npar])
                    pltpu.async_copy(
                        cumk_hbm.at[pl.ds((src + 1) * _FHW, _FHW)],
                        fk_v.at[pl.ds(npar * _FHW, _FHW)], sem_f.at[npar])

                @pl.when(i == 0)
                def _first():
                    plsc.parallel_loop(0, _NGRP, 1, unroll=1)(
                        make_group_body(0, 0, True))

                @pl.when(i > 0)
                def _rest():
                    plsc.parallel_loop(0, _NGRP, 1, unroll=1)(
                        make_group_body(ibase, fbase, False))

                return _

            lax.fori_loop(0, nk, k_body, 0)

            pltpu.sync_copy(acc_v, out_hbm.at[pl.ds(dest * _CHW, _CHW)])

    plsc.subcore_barrier()
    mslot = _sel(wid, [_MERGE[i][0] for i in range(_NW)])
    pslot = _sel(wid, [_MERGE[i][1] for i in range(_NW)])
    pslot2 = _sel(wid, [_MERGE[i][2] for i in range(_NW)])

    @pl.when(mslot >= 0)
    def _merge_job():
        pltpu.async_copy(out_hbm.at[pl.ds(mslot * _CHW, _CHW)],
                         img_v.at[pl.ds(0, _CHW)], sem_i.at[0])
        pltpu.async_copy(out_hbm.at[pl.ds(pslot * _CHW, _CHW)],
                         img_v.at[pl.ds(_CHW, _CHW)], sem_i.at[1])
        pltpu.make_async_copy(out_hbm.at[pl.ds(mslot * _CHW, _CHW)],
                              img_v.at[pl.ds(0, _CHW)], sem_i.at[0]).wait()
        pltpu.make_async_copy(out_hbm.at[pl.ds(pslot * _CHW, _CHW)],
                              img_v.at[pl.ds(_CHW, _CHW)], sem_i.at[1]).wait()

        @plsc.parallel_loop(0, _CHW // 16, 1, unroll=4)
        def _madd(i):
            gl = i * 16
            img_v[pl.ds(gl, 16)] = (img_v[pl.ds(gl, 16)]
                                    + img_v[pl.ds(_CHW + gl, 16)])

        @pl.when(pslot2 >= 0)
        def _second_partial():
            pltpu.sync_copy(out_hbm.at[pl.ds(pslot2 * _CHW, _CHW)],
                            img_v.at[pl.ds(_CHW, _CHW)])

            @plsc.parallel_loop(0, _CHW // 16, 1, unroll=4)
            def _madd2(i):
                gl = i * 16
                img_v[pl.ds(gl, 16)] = (img_v[pl.ds(gl, 16)]
                                        + img_v[pl.ds(_CHW + gl, 16)])

        pltpu.sync_copy(img_v.at[pl.ds(0, _CHW)],
                        out_hbm.at[pl.ds(mslot * _CHW, _CHW)])


_mesh = plsc.VectorSubcoreMesh(core_axis_name="c", subcore_axis_name="s",
                               num_cores=_NC, num_subcores=_NS)

_sc_call = functools.partial(
    pl.kernel,
    mesh=_mesh,
    compiler_params=pltpu.CompilerParams(use_tc_tiling_on_sc=False,
                                         needs_layout_passes=False),
    out_type=jax.ShapeDtypeStruct((_NOUT * _CHW,), jnp.float32),
    scratch_types=[
        pltpu.VMEM((2 * _CHW,), jnp.float32),
        pltpu.VMEM((_CHW,), jnp.float32),
        pltpu.VMEM((_FHW,), jnp.float32),
        pltpu.VMEM((2 * _FHW,), jnp.float32),
        pltpu.SemaphoreType.DMA((2,)),
        pltpu.SemaphoreType.DMA((2,)),
    ],
)(_sc_body)


def kernel(flows, images):
    dtype = flows.dtype
    cum = jnp.cumsum(flows.astype(jnp.float32), axis=1).astype(dtype)
    sy = 2.0 / _H
    sx = 2.0 / _W
    gyc = jnp.linspace(-1.0 + sy * 0.5, 1.0 - sy * 0.5, _H, dtype=dtype)
    gxc = jnp.linspace(-1.0 + sx * 0.5, 1.0 - sx * 0.5, _W, dtype=dtype)
    bx = jnp.tile(gxc, _H)
    by = jnp.repeat(gyc, _W)
    base = jnp.stack([bx, by])
    cumf = cum.reshape(_B, _L, 2, _HW)
    cumt = (cumf + base[None, None]).reshape(-1)
    out_flat = _sc_call(images.reshape(-1), cumt, cumf.reshape(-1))
    return out_flat[: _NSLOT * _CHW].reshape(_B, _L, _C, _H, _W)
